# Initial kernel scaffold; baseline (speedup 1.0000x reference)
import jax
import jax.numpy as jnp
from jax.experimental import pallas as pl


def kernel(x, edge_index, edge_attr, We, be, W1, b1, gamma, beta, W2, b2, eps):
    raise NotImplementedError("write your pallas kernel here")


# speedup vs baseline: 1.6876x; 1.6876x over previous
import functools

import jax
import jax.numpy as jnp
from jax import lax
from jax.experimental import pallas as pl
from jax.experimental.pallas import tpu as pltpu
from jax.experimental.pallas import tpu_sc as plsc

N = 10000
E = 320000
D = 128
ED = 16
H = D // 2

NC = 2
NS = 16
L = 16
B = 128
NB = 160
EPT = NB * B
EPAD = NS * EPT
NPAD = 10112
SLAB = NPAD // NS


def _rea_body(ea_ref, we_ref, be_ref, out_ref):
    out_ref[0] = (
        jnp.dot(ea_ref[...], we_ref[0], preferred_element_type=jnp.float32)
        + be_ref[0]
    )


_REA_MB = 2048


def _rea_call(ea_pad, We_t, be_t):
    return pl.pallas_call(
        _rea_body,
        grid=(NC, EPAD // _REA_MB),
        in_specs=[
            pl.BlockSpec((_REA_MB, ED), lambda i, j: (j, 0)),
            pl.BlockSpec((1, ED, H), lambda i, j: (i, 0, 0)),
            pl.BlockSpec((1, 1, H), lambda i, j: (i, 0, 0)),
        ],
        out_specs=pl.BlockSpec((1, _REA_MB, H), lambda i, j: (i, j, 0)),
        out_shape=jax.ShapeDtypeStruct((NC, EPAD, H), jnp.float32),
    )(ea_pad, We_t, be_t)


_SC_MESH = plsc.VectorSubcoreMesh(core_axis_name="c", subcore_axis_name="s")


@functools.partial(
    pl.kernel,
    out_type=jax.ShapeDtypeStruct((NC, NPAD, H), jnp.float32),
    mesh=_SC_MESH,
    scratch_types=[
        pltpu.VMEM((NB, B), jnp.int32),
        pltpu.VMEM((NB, B), jnp.int32),
        pltpu.VMEM((B, H), jnp.float32),
        pltpu.VMEM((B, H), jnp.float32),
        pltpu.VMEM((B, H), jnp.float32),
        pltpu.VMEM((B, H), jnp.float32),
        pltpu.VMEM((B, H), jnp.float32),
        pltpu.VMEM((B, H), jnp.float32),
        pltpu.VMEM_SHARED((NPAD, H), jnp.float32),
        pltpu.SemaphoreType.DMA,
        pltpu.SemaphoreType.DMA,
        pltpu.SemaphoreType.DMA,
        pltpu.SemaphoreType.DMA,
        pltpu.SemaphoreType.DMA,
        pltpu.SemaphoreType.DMA,
    ],
    compiler_params=pltpu.CompilerParams(use_tc_tiling_on_sc=False),
)
def _sc_scatter(x_hbm, rea_hbm, src_hbm, dst_hbm, zer_hbm, part_hbm,
                src_v, dst_v, g0, g1, r0, r1, m0, m1, acc,
                gs0, gs1, rs0, rs1, ss0, ss1):
    c = lax.axis_index("c")
    s = lax.axis_index("s")

    pltpu.sync_copy(src_hbm.at[s], src_v)
    pltpu.sync_copy(dst_hbm.at[s], dst_v)
    pltpu.sync_copy(zer_hbm, acc.at[pl.ds(s * SLAB, SLAB)])
    plsc.subcore_barrier()

    ebase = s * EPT
    xh = x_hbm.at[c]
    gbufs = (g0, g1)
    rbufs = (r0, r1)
    mbufs = (m0, m1)
    gsems = (gs0, gs1)
    rsems = (rs0, rs1)
    ssems = (ss0, ss1)

    for k in range(2):
        pltpu.async_copy(xh.at[src_v.at[k]], gbufs[k], gsems[k])
        pltpu.async_copy(
            rea_hbm.at[c, pl.ds(ebase + k * B, B)], rbufs[k], rsems[k])

    def outer(jo, carry):
        for k in range(2):
            j = jo * 2 + k
            g, r, m = gbufs[k], rbufs[k], mbufs[k]
            pltpu.make_async_copy(xh.at[src_v.at[j]], g, gsems[k]).wait()
            pltpu.make_async_copy(
                rea_hbm.at[c, pl.ds(ebase + j * B, B)], r, rsems[k]).wait()

            @pl.when(jo > 0)
            def _():
                pltpu.make_async_copy(m, acc.at[dst_v.at[j]], ssems[k]).wait()

            def edge(e, carry2):
                for cc in range(H // L):
                    sl = pl.ds(cc * L, L)
                    m[e, sl] = jnp.maximum(g[e, sl] + r[e, sl], 0.0)
                return carry2

            lax.fori_loop(0, B, edge, 0)

            pltpu.async_copy(m, acc.at[dst_v.at[j]], ssems[k], add=True)

            @pl.when(jo < NB // 2 - 1)
            def _():
                jn = j + 2
                pltpu.async_copy(xh.at[src_v.at[jn]], g, gsems[k])
                pltpu.async_copy(
                    rea_hbm.at[c, pl.ds(ebase + jn * B, B)], r, rsems[k])
        return carry

    lax.fori_loop(0, NB // 2, outer, 0)

    for k in range(2):
        pltpu.make_async_copy(
            mbufs[k], acc.at[dst_v.at[NB - 2 + k]], ssems[k]).wait()
    plsc.subcore_barrier()
    pltpu.sync_copy(acc.at[pl.ds(s * SLAB, SLAB)],
                    part_hbm.at[c, pl.ds(s * SLAB, SLAB)])


def _mlp_body(sc_ref, x_ref, p_ref, w1_ref, b1_ref, gm_ref, bt_ref,
              w2_ref, b2_ref, o_ref):
    mr = jnp.concatenate([p_ref[0], p_ref[1]], axis=-1)
    h = x_ref[...] * sc_ref[0] + mr
    h = jnp.dot(h, w1_ref[...], preferred_element_type=jnp.float32) + b1_ref[...]
    mu = jnp.mean(h, axis=-1, keepdims=True)
    var = jnp.mean((h - mu) ** 2, axis=-1, keepdims=True)
    h = (h - mu) * lax.rsqrt(var + 1e-5) * gm_ref[...] + bt_ref[...]
    h = jnp.maximum(h, 0.0)
    o_ref[...] = (
        jnp.dot(h, w2_ref[...], preferred_element_type=jnp.float32) + b2_ref[...]
    )


_MLP_R = 1000


def _mlp_call(scale, x, parts, W1, b1_row, gm_row, bt_row, W2, b2_row):
    return pl.pallas_call(
        _mlp_body,
        grid=(N // _MLP_R,),
        in_specs=[
            pl.BlockSpec(memory_space=pltpu.SMEM),
            pl.BlockSpec((_MLP_R, D), lambda i: (i, 0)),
            pl.BlockSpec((NC, _MLP_R, H), lambda i: (0, i, 0)),
            pl.BlockSpec((D, 2 * D), lambda i: (0, 0)),
            pl.BlockSpec((1, 2 * D), lambda i: (0, 0)),
            pl.BlockSpec((1, 2 * D), lambda i: (0, 0)),
            pl.BlockSpec((1, 2 * D), lambda i: (0, 0)),
            pl.BlockSpec((2 * D, D), lambda i: (0, 0)),
            pl.BlockSpec((1, D), lambda i: (0, 0)),
        ],
        out_specs=pl.BlockSpec((_MLP_R, D), lambda i: (i, 0)),
        out_shape=jax.ShapeDtypeStruct((N, D), jnp.float32),
    )(scale, x, parts, W1, b1_row, gm_row, bt_row, W2, b2_row)


def kernel(x, edge_index, edge_attr, We, be, W1, b1, gamma, beta, W2, b2, eps):
    ei = edge_index.astype(jnp.int32)
    src = ei[1]
    dst = ei[0]
    pad = EPAD - E
    src_p = jnp.concatenate(
        [src, jnp.zeros((pad,), jnp.int32)]).reshape(NS, NB, B)
    dst_p = jnp.concatenate(
        [dst, jnp.full((pad,), N, jnp.int32)]).reshape(NS, NB, B)
    ea_p = jnp.concatenate(
        [edge_attr, jnp.zeros((pad, ED), edge_attr.dtype)], axis=0)
    xt = x.reshape(N, NC, H).transpose(1, 0, 2)

    We_t = We.reshape(ED, NC, H).transpose(1, 0, 2)
    be_t = be.reshape(NC, 1, H)
    rea = _rea_call(ea_p, We_t, be_t)
    zer = jnp.zeros((SLAB, H), jnp.float32)
    parts = _sc_scatter(xt, rea, src_p, dst_p, zer)

    scale = (1.0 + eps).astype(jnp.float32).reshape(1)
    out = _mlp_call(scale, x, parts[:, :N, :], W1,
                    b1.reshape(1, 2 * D), gamma.reshape(1, 2 * D),
                    beta.reshape(1, 2 * D), W2, b2.reshape(1, D))
    return out
